# baseline (speedup 1.0000x reference)
import jax
import jax.numpy as jnp
from jax.experimental import pallas as pl
from jax.experimental.pallas import tpu as pltpu

_B, _N2, _N1, _F, _A, _O, _L = 1024, 10, 25, 256, 128, 256, 50
_N2P = 16
_BB = 32
_STEPS = _B // _BB
_R = _BB * _N2 * _N1
_GP = _BB * _N2P
_CB = 64


def _l2n(x):
    return x * jax.lax.rsqrt(jnp.maximum(jnp.sum(x * x, axis=-1, keepdims=True), 1e-12))


def _fused(x0_ref, x1_ref, x2_ref, wagg0_ref, w0s_ref, w0a_ref, wagg1_ref,
           w1s_ref, w1a_ref, wcls_ref, out_ref, s1_ref, agg0_ref):
    i = pl.program_id(0)

    @pl.when(i == 0)
    def _build_s1():
        r = jax.lax.broadcasted_iota(jnp.int32, (_GP, _R), 0)
        j = jax.lax.broadcasted_iota(jnp.int32, (_GP, _R), 1)
        grp = (r // _N2P) * _N2 + jax.lax.rem(r, _N2P)
        ok = (jax.lax.rem(r, _N2P) < _N2) & (grp == j // _N1)
        s1_ref[...] = jnp.where(ok, 1.0, 0.0).astype(jnp.bfloat16)

    x2 = x2_ref[...].astype(jnp.bfloat16)
    t = jnp.maximum(jnp.dot(x2, wagg0_ref[...], preferred_element_type=jnp.float32), 0.0)
    agg0 = jnp.dot(s1_ref[...], t.astype(jnp.bfloat16),
                   preferred_element_type=jnp.float32) * (1.0 / _N1)
    agg0_ref[pl.ds(i * _GP, _GP), :] = agg0.astype(jnp.bfloat16)

    @pl.when(i % 2 == 1)
    def _tail():
        k = i // 2
        x1 = jnp.pad(x1_ref[...], ((0, 0), (0, _N2P - _N2), (0, 0)))
        x1 = x1.reshape(_CB * _N2P, _F).astype(jnp.bfloat16)
        a0 = agg0_ref[pl.ds(k * _CB * _N2P, _CB * _N2P), :]
        h1 = jnp.maximum(
            jnp.dot(x1, w0s_ref[...], preferred_element_type=jnp.float32)
            + jnp.dot(a0, w0a_ref[...], preferred_element_type=jnp.float32), 0.0)
        h1 = _l2n(h1).astype(jnp.bfloat16)
        g = jnp.maximum(jnp.dot(h1, wagg1_ref[...], preferred_element_type=jnp.float32), 0.0)
        agg1 = jnp.sum(g.reshape(_CB, _N2P, _A), axis=1) * (1.0 / _N2)
        h0 = (jnp.dot(x0_ref[...], w1s_ref[...], preferred_element_type=jnp.float32)
              + jnp.dot(agg1, w1a_ref[...], preferred_element_type=jnp.float32))
        h0 = _l2n(_l2n(h0))
        out_ref[...] = jnp.maximum(
            jnp.dot(h0, wcls_ref[...], preferred_element_type=jnp.float32), 0.0)


def _full(shape):
    return pl.BlockSpec(shape, lambda i: (0,) * len(shape))


def kernel(x0, x1, x2, Wagg0, Wagg1, Wcomb0, Wcomb1, Wcls):
    w0s = Wcomb0[:_F].astype(jnp.bfloat16)
    w0a = Wcomb0[_F:].astype(jnp.bfloat16)
    w1s, w1a = Wcomb1[:_F], Wcomb1[_F:]
    x2r = x2.reshape(_B * _N2 * _N1, _F)
    return pl.pallas_call(
        _fused,
        grid=(_STEPS,),
        in_specs=[
            pl.BlockSpec((_CB, _F), lambda i: (i // 2, 0)),
            pl.BlockSpec((_CB, _N2, _F), lambda i: (i // 2, 0, 0)),
            pl.BlockSpec((_R, _F), lambda i: (i, 0)),
            _full((_F, _A)), _full((_F, _O)), _full((_A, _O)),
            _full((_O, _A)), _full((_F, _O)), _full((_A, _O)),
            _full((_O, _L)),
        ],
        out_specs=pl.BlockSpec((_CB, _L), lambda i: (i // 2, 0)),
        out_shape=jax.ShapeDtypeStruct((_B, _L), jnp.float32),
        scratch_shapes=[
            pltpu.VMEM((_GP, _R), jnp.bfloat16),
            pltpu.VMEM((_B * _N2P, _A), jnp.bfloat16),
        ],
        compiler_params=pltpu.CompilerParams(dimension_semantics=("arbitrary",)),
    )(x0, x1, x2r, Wagg0.astype(jnp.bfloat16), w0s, w0a,
      Wagg1.astype(jnp.bfloat16), w1s, w1a, Wcls)


# speedup vs baseline: 1.9894x; 1.9894x over previous
import jax
import jax.numpy as jnp
from jax.experimental import pallas as pl
from jax.experimental.pallas import tpu as pltpu

_B, _N2, _N1, _F, _A, _O, _L = 1024, 10, 25, 256, 128, 256, 50
_N1A = 24
_N2P = 16
_BB = 32
_STEPS = _B // _BB
_CB = 64
_S = 8
_SB = _BB // _S


def _l2n(x):
    return x * jax.lax.rsqrt(jnp.maximum(jnp.sum(x * x, axis=-1, keepdims=True), 1e-12))


def _x2_copies(x2_hbm, bulk_ref, last_ref, sems, step, slot):
    copies = []
    for s in range(_S):
        rows = pl.ds(step * _BB + s * _SB, _SB)
        dst = pl.ds(s * _SB, _SB)
        copies.append(pltpu.make_async_copy(
            x2_hbm.at[rows, :, pl.ds(0, _N1A), :],
            bulk_ref.at[slot, dst],
            sems.at[slot, s]))
        copies.append(pltpu.make_async_copy(
            x2_hbm.at[rows, :, pl.ds(_N1A, 1), :],
            last_ref.at[slot, dst],
            sems.at[slot, _S + s]))
    return copies


def _fused(x0_ref, x1_ref, x2_hbm, wagg0_ref, w0s_ref, w0a_ref, wagg1_ref,
           w1s_ref, w1a_ref, wcls_ref, out_ref, bulk_ref, last_ref, agg0_ref,
           sems):
    i = pl.program_id(0)
    slot = jax.lax.rem(i, 3)

    @pl.when(i == 0)
    def _prologue():
        for c in _x2_copies(x2_hbm, bulk_ref, last_ref, sems, 0, 0):
            c.start()
        for c in _x2_copies(x2_hbm, bulk_ref, last_ref, sems, 1, 1):
            c.start()

    @pl.when(i + 2 < _STEPS)
    def _prefetch():
        nxt = jax.lax.rem(i + 2, 3)
        for c in _x2_copies(x2_hbm, bulk_ref, last_ref, sems, i + 2, nxt):
            c.start()

    for c in _x2_copies(x2_hbm, bulk_ref, last_ref, sems, i, slot):
        c.wait()

    x2 = bulk_ref[slot].reshape(_BB * _N2 * _N1A, _F).astype(jnp.bfloat16)
    t = jnp.maximum(jnp.dot(x2, wagg0_ref[...], preferred_element_type=jnp.float32), 0.0)
    s24 = jnp.sum(t.reshape(_BB, _N2, _N1A, _A), axis=2)
    v24 = jnp.pad(last_ref[slot, :, :, 0, :], ((0, 0), (0, _N2P - _N2), (0, 0)))
    v24 = v24.reshape(_BB * _N2P, _F).astype(jnp.bfloat16)
    t24 = jnp.maximum(jnp.dot(v24, wagg0_ref[...], preferred_element_type=jnp.float32), 0.0)
    s24 = jnp.pad(s24, ((0, 0), (0, _N2P - _N2), (0, 0))).reshape(_BB * _N2P, _A)
    agg0 = (s24 + t24) * (1.0 / _N1)
    agg0_ref[pl.ds(i * _BB * _N2P, _BB * _N2P), :] = agg0.astype(jnp.bfloat16)

    @pl.when(i % 2 == 1)
    def _tail():
        k = i // 2
        x1 = jnp.pad(x1_ref[...], ((0, 0), (0, _N2P - _N2), (0, 0)))
        x1 = x1.reshape(_CB * _N2P, _F).astype(jnp.bfloat16)
        a0 = agg0_ref[pl.ds(k * _CB * _N2P, _CB * _N2P), :]
        h1 = jnp.maximum(
            jnp.dot(x1, w0s_ref[...], preferred_element_type=jnp.float32)
            + jnp.dot(a0, w0a_ref[...], preferred_element_type=jnp.float32), 0.0)
        h1 = _l2n(h1).astype(jnp.bfloat16)
        g = jnp.maximum(jnp.dot(h1, wagg1_ref[...], preferred_element_type=jnp.float32), 0.0)
        agg1 = jnp.sum(g.reshape(_CB, _N2P, _A), axis=1) * (1.0 / _N2)
        h0 = (jnp.dot(x0_ref[...], w1s_ref[...], preferred_element_type=jnp.float32)
              + jnp.dot(agg1, w1a_ref[...], preferred_element_type=jnp.float32))
        h0 = _l2n(_l2n(h0))
        out_ref[...] = jnp.maximum(
            jnp.dot(h0, wcls_ref[...], preferred_element_type=jnp.float32), 0.0)


def _full(shape):
    return pl.BlockSpec(shape, lambda i: (0,) * len(shape))


def kernel(x0, x1, x2, Wagg0, Wagg1, Wcomb0, Wcomb1, Wcls):
    w0s = Wcomb0[:_F].astype(jnp.bfloat16)
    w0a = Wcomb0[_F:].astype(jnp.bfloat16)
    w1s, w1a = Wcomb1[:_F], Wcomb1[_F:]
    return pl.pallas_call(
        _fused,
        grid=(_STEPS,),
        in_specs=[
            pl.BlockSpec((_CB, _F), lambda i: (i // 2, 0)),
            pl.BlockSpec((_CB, _N2, _F), lambda i: (i // 2, 0, 0)),
            pl.BlockSpec(memory_space=pl.ANY),
            _full((_F, _A)), _full((_F, _O)), _full((_A, _O)),
            _full((_O, _A)), _full((_F, _O)), _full((_A, _O)),
            _full((_O, _L)),
        ],
        out_specs=pl.BlockSpec((_CB, _L), lambda i: (i // 2, 0)),
        out_shape=jax.ShapeDtypeStruct((_B, _L), jnp.float32),
        scratch_shapes=[
            pltpu.VMEM((3, _BB, _N2, _N1A, _F), jnp.float32),
            pltpu.VMEM((3, _BB, _N2, 1, _F), jnp.float32),
            pltpu.VMEM((_B * _N2P, _A), jnp.bfloat16),
            pltpu.SemaphoreType.DMA((3, 2 * _S)),
        ],
        compiler_params=pltpu.CompilerParams(dimension_semantics=("arbitrary",)),
    )(x0, x1, x2, Wagg0.astype(jnp.bfloat16), w0s, w0a,
      Wagg1.astype(jnp.bfloat16), w1s, w1a, Wcls)


# speedup vs baseline: 1.9922x; 1.0014x over previous
import jax
import jax.numpy as jnp
from jax.experimental import pallas as pl
from jax.experimental.pallas import tpu as pltpu

_B, _N2, _N1, _F, _A, _O, _L = 1024, 10, 25, 256, 128, 256, 50
_N1A = 24
_N2P = 16
_BB = 32
_STEPS = _B // _BB
_CB = 64
_S = 8
_SB = _BB // _S


def _l2n(x):
    return x * jax.lax.rsqrt(jnp.maximum(jnp.sum(x * x, axis=-1, keepdims=True), 1e-12))


def _x2_copies(x2_hbm, bulk_ref, last_ref, sems, step, slot):
    copies = []
    for s in range(_S):
        rows = pl.ds(step * _BB + s * _SB, _SB)
        dst = pl.ds(s * _SB, _SB)
        copies.append(pltpu.make_async_copy(
            x2_hbm.at[rows, :, pl.ds(0, _N1A), :],
            bulk_ref.at[slot, dst],
            sems.at[slot, s]))
        copies.append(pltpu.make_async_copy(
            x2_hbm.at[rows, :, pl.ds(_N1A, 1), :],
            last_ref.at[slot, dst],
            sems.at[slot, _S + s]))
    return copies


def _fused(x0_ref, x1_ref, x2_hbm, wagg0_ref, w0s_ref, w0a_ref, wagg1_ref,
           w1s_ref, w1a_ref, wcls_ref, out_ref, bulk_ref, last_ref, agg0_ref,
           sems):
    i = pl.program_id(0)
    slot = jax.lax.rem(i, 4)

    @pl.when(i == 0)
    def _prologue():
        for step in range(3):
            for c in _x2_copies(x2_hbm, bulk_ref, last_ref, sems, step, step):
                c.start()

    @pl.when(i + 3 < _STEPS)
    def _prefetch():
        nxt = jax.lax.rem(i + 3, 4)
        for c in _x2_copies(x2_hbm, bulk_ref, last_ref, sems, i + 3, nxt):
            c.start()

    for c in _x2_copies(x2_hbm, bulk_ref, last_ref, sems, i, slot):
        c.wait()

    x2 = bulk_ref[slot].reshape(_BB * _N2 * _N1A, _F).astype(jnp.bfloat16)
    t = jnp.maximum(jnp.dot(x2, wagg0_ref[...], preferred_element_type=jnp.float32), 0.0)
    s24 = jnp.sum(t.reshape(_BB, _N2, _N1A, _A), axis=2)
    v24 = jnp.pad(last_ref[slot, :, :, 0, :], ((0, 0), (0, _N2P - _N2), (0, 0)))
    v24 = v24.reshape(_BB * _N2P, _F).astype(jnp.bfloat16)
    t24 = jnp.maximum(jnp.dot(v24, wagg0_ref[...], preferred_element_type=jnp.float32), 0.0)
    s24 = jnp.pad(s24, ((0, 0), (0, _N2P - _N2), (0, 0))).reshape(_BB * _N2P, _A)
    agg0 = (s24 + t24) * (1.0 / _N1)
    agg0_ref[pl.ds(i * _BB * _N2P, _BB * _N2P), :] = agg0.astype(jnp.bfloat16)

    @pl.when(i % 2 == 1)
    def _tail():
        k = i // 2
        x1 = jnp.pad(x1_ref[...], ((0, 0), (0, _N2P - _N2), (0, 0)))
        x1 = x1.reshape(_CB * _N2P, _F).astype(jnp.bfloat16)
        a0 = agg0_ref[pl.ds(k * _CB * _N2P, _CB * _N2P), :]
        h1 = jnp.maximum(
            jnp.dot(x1, w0s_ref[...], preferred_element_type=jnp.float32)
            + jnp.dot(a0, w0a_ref[...], preferred_element_type=jnp.float32), 0.0)
        h1 = _l2n(h1).astype(jnp.bfloat16)
        g = jnp.maximum(jnp.dot(h1, wagg1_ref[...], preferred_element_type=jnp.float32), 0.0)
        agg1 = jnp.sum(g.reshape(_CB, _N2P, _A), axis=1) * (1.0 / _N2)
        h0 = (jnp.dot(x0_ref[...], w1s_ref[...], preferred_element_type=jnp.float32)
              + jnp.dot(agg1, w1a_ref[...], preferred_element_type=jnp.float32))
        h0 = _l2n(_l2n(h0))
        out_ref[...] = jnp.maximum(
            jnp.dot(h0, wcls_ref[...], preferred_element_type=jnp.float32), 0.0)


def _full(shape):
    return pl.BlockSpec(shape, lambda i: (0,) * len(shape))


def kernel(x0, x1, x2, Wagg0, Wagg1, Wcomb0, Wcomb1, Wcls):
    w0s = Wcomb0[:_F].astype(jnp.bfloat16)
    w0a = Wcomb0[_F:].astype(jnp.bfloat16)
    w1s, w1a = Wcomb1[:_F], Wcomb1[_F:]
    return pl.pallas_call(
        _fused,
        grid=(_STEPS,),
        in_specs=[
            pl.BlockSpec((_CB, _F), lambda i: (i // 2, 0)),
            pl.BlockSpec((_CB, _N2, _F), lambda i: (i // 2, 0, 0)),
            pl.BlockSpec(memory_space=pl.ANY),
            _full((_F, _A)), _full((_F, _O)), _full((_A, _O)),
            _full((_O, _A)), _full((_F, _O)), _full((_A, _O)),
            _full((_O, _L)),
        ],
        out_specs=pl.BlockSpec((_CB, _L), lambda i: (i // 2, 0)),
        out_shape=jax.ShapeDtypeStruct((_B, _L), jnp.float32),
        scratch_shapes=[
            pltpu.VMEM((4, _BB, _N2, _N1A, _F), jnp.float32),
            pltpu.VMEM((4, _BB, _N2, 1, _F), jnp.float32),
            pltpu.VMEM((_B * _N2P, _A), jnp.bfloat16),
            pltpu.SemaphoreType.DMA((4, 2 * _S)),
        ],
        compiler_params=pltpu.CompilerParams(dimension_semantics=("arbitrary",)),
    )(x0, x1, x2, Wagg0.astype(jnp.bfloat16), w0s, w0a,
      Wagg1.astype(jnp.bfloat16), w1s, w1a, Wcls)
